# Initial kernel scaffold; baseline (speedup 1.0000x reference)
import jax
import jax.numpy as jnp
from jax.experimental import pallas as pl


def kernel(x, W):
    raise NotImplementedError("write your pallas kernel here")


# speedup vs baseline: 6.6198x; 6.6198x over previous
import functools

import jax
import jax.numpy as jnp
from jax.experimental import pallas as pl
from jax.experimental.pallas import tpu as pltpu
from jax.experimental.pallas import tpu_sc as plsc

_NUM_E = 8192
_E_DIM = 256
_BETA = 0.25
_N_TOK = 4096
_BLK = 256
_N_BLKS = _N_TOK // _BLK
_GW = 128


def _vq_block(x_ref, wt_ref, enc_ref, idx_ref, loss_ref, perp_ref,
              cnt_ref, acc_ref):
    r = pl.program_id(0)
    x = x_ref[...]
    wt = wt_ref[...]
    s = jax.lax.dot_general(x, wt, (((1,), (0,)), ((), ())),
                            preferred_element_type=jnp.float32)
    w2 = jnp.sum(wt * wt, axis=0, keepdims=True)
    x2 = jnp.sum(x * x, axis=1, keepdims=True)
    d = (w2 + x2) - 2.0 * s
    minv = jnp.min(d, axis=1, keepdims=True)
    col = jax.lax.broadcasted_iota(jnp.int32, (_BLK, _NUM_E), 1)
    idx = jnp.min(jnp.where(d == minv, col, _NUM_E), axis=1, keepdims=True)
    enc = jnp.where(col == idx, 1.0, 0.0).astype(jnp.float32)
    enc_ref[...] = enc
    idx_ref[...] = idx.reshape(1, _BLK, 1)

    bcnt = jnp.sum(enc, axis=0, keepdims=True)
    bloss = jnp.sum(minv)

    @pl.when(r == 0)
    def _():
        cnt_ref[...] = bcnt
        acc_ref[0] = bloss

    @pl.when(r > 0)
    def _():
        cnt_ref[...] += bcnt
        acc_ref[0] += bloss

    @pl.when(r == _N_BLKS - 1)
    def _():
        loss_ref[...] = jnp.reshape(
            acc_ref[0] * ((1.0 + _BETA) / (_N_TOK * _E_DIM)), (1, 1))
        e_mean = cnt_ref[...] * (1.0 / _N_TOK)
        ent = jnp.sum(e_mean * jnp.log(e_mean + 1e-10))
        perp_ref[...] = jnp.reshape(jnp.exp(-ent), (1, 1))


def _vq_pallas(x_flat, Wt):
    return pl.pallas_call(
        _vq_block,
        grid=(_N_BLKS,),
        in_specs=[
            pl.BlockSpec((_BLK, _E_DIM), lambda r: (r, 0)),
            pl.BlockSpec((_E_DIM, _NUM_E), lambda r: (0, 0)),
        ],
        out_specs=[
            pl.BlockSpec((_BLK, _NUM_E), lambda r: (r, 0)),
            pl.BlockSpec((1, _BLK, 1), lambda r: (r, 0, 0)),
            pl.BlockSpec((1, 1), lambda r: (0, 0)),
            pl.BlockSpec((1, 1), lambda r: (0, 0)),
        ],
        out_shape=[
            jax.ShapeDtypeStruct((_N_TOK, _NUM_E), jnp.float32),
            jax.ShapeDtypeStruct((_N_BLKS, _BLK, 1), jnp.int32),
            jax.ShapeDtypeStruct((1, 1), jnp.float32),
            jax.ShapeDtypeStruct((1, 1), jnp.float32),
        ],
        scratch_shapes=[
            pltpu.VMEM((1, _NUM_E), jnp.float32),
            pltpu.SMEM((1,), jnp.float32),
        ],
    )(x_flat, Wt)


def _gather_rows(W, idx_flat):
    @functools.partial(
        pl.kernel,
        out_type=jax.ShapeDtypeStruct((_N_TOK, _E_DIM), jnp.float32),
        mesh=plsc.VectorSubcoreMesh(core_axis_name="core",
                                    subcore_axis_name="subcore"),
    )
    def k(w_hbm, i_hbm, o_hbm):
        def body(i_vmem, o_vmem):
            pltpu.sync_copy(w_hbm.at[i_vmem.at[0]], o_vmem)

        pltpu.emit_pipeline(
            body,
            grid=(_N_TOK // _GW,),
            in_specs=[pl.BlockSpec((1, _GW), index_map=lambda i: (0, i))],
            out_specs=[pl.BlockSpec((_GW, _E_DIM), index_map=lambda i: (i, 0))],
            core_axis_name=("core", "subcore"),
            dimension_semantics=(pltpu.PARALLEL,),
        )(i_hbm, o_hbm)

    return k(W, idx_flat)


def kernel(x, W):
    b, c, h, w = x.shape
    x_p = jnp.transpose(x, (0, 2, 3, 1))
    x_flat = x_p.reshape(-1, _E_DIM)
    enc, idx, loss, perp = _vq_pallas(x_flat, W.T)
    xq_flat = _gather_rows(W, idx.reshape(1, _N_TOK))
    x_q = jnp.transpose(xq_flat.reshape(b, h, w, c), (0, 3, 1, 2))
    return loss[0, 0], x_q, perp[0, 0], enc
